# Initial kernel scaffold; baseline (speedup 1.0000x reference)
import jax
import jax.numpy as jnp
from jax.experimental import pallas as pl


def kernel(global_rotation, transl, hand_pose, obj_points, base_verts, pose_basis, faces, sample_idx):
    raise NotImplementedError("write your pallas kernel here")


# speedup vs baseline: 2.8603x; 2.8603x over previous
import functools

import jax
import jax.numpy as jnp
from jax.experimental import pallas as pl

_B = 2
_V = 778
_F = 512
_S = 512
_N = 8192
_TN = 512


def _rodrigues(r):
    theta = jnp.linalg.norm(r) + 1e-8
    k = r / theta
    K = jnp.array([[0.0, -k[2], k[1]],
                   [k[2], 0.0, -k[0]],
                   [-k[1], k[0], 0.0]], dtype=r.dtype)
    I = jnp.eye(3, dtype=r.dtype)
    return I + jnp.sin(theta) * K + (1.0 - jnp.cos(theta)) * (K @ K)


def _sr_body(obj_ref, sr_ref, face_ref, pen_ref, cmap_ref):
    ox = obj_ref[:, 0:1]
    oy = obj_ref[:, 1:2]
    oz = obj_ref[:, 2:3]
    srx = sr_ref[0, 0:1, :]
    sry = sr_ref[0, 1:2, :]
    srz = sr_ref[0, 2:3, :]

    dx = ox - srx
    dy = oy - sry
    dz = oz - srz
    d2 = dx * dx + dy * dy + dz * dz
    d2min = jnp.min(d2, axis=1, keepdims=True)
    lane = jax.lax.broadcasted_iota(jnp.int32, (_TN, _S), 1)
    idx = jnp.min(jnp.where(d2 == d2min, lane, _S), axis=1, keepdims=True)
    onehot = lane == idx
    zero = jnp.zeros((), jnp.float32)
    tx = jnp.sum(jnp.where(onehot, srx, zero), axis=1, keepdims=True)
    ty = jnp.sum(jnp.where(onehot, sry, zero), axis=1, keepdims=True)
    tz = jnp.sum(jnp.where(onehot, srz, zero), axis=1, keepdims=True)

    cmap_ref[:, :] = 1.0 - 2.0 * (jax.nn.sigmoid(100.0 * d2min) - 0.5)

    rdx = tx - ox
    rdy = ty - oy
    rdz = tz - oz
    v0x = face_ref[0, 0:1, :]
    v0y = face_ref[0, 1:2, :]
    v0z = face_ref[0, 2:3, :]
    e1x = face_ref[0, 3:4, :]
    e1y = face_ref[0, 4:5, :]
    e1z = face_ref[0, 5:6, :]
    e2x = face_ref[0, 6:7, :]
    e2y = face_ref[0, 7:8, :]
    e2z = face_ref[0, 8:9, :]

    hx = rdy * e2z - rdz * e2y
    hy = rdz * e2x - rdx * e2z
    hz = rdx * e2y - rdy * e2x
    a = e1x * hx + e1y * hy + e1z * hz
    aok = jnp.abs(a) > 1e-9
    f = 1.0 / jnp.where(jnp.abs(a) < 1e-9, 1e-9, a)
    sx = ox - v0x
    sy = oy - v0y
    sz = oz - v0z
    u = f * (sx * hx + sy * hy + sz * hz)
    qx = sy * e1z - sz * e1y
    qy = sz * e1x - sx * e1z
    qz = sx * e1y - sy * e1x
    vv = f * (rdx * qx + rdy * qy + rdz * qz)
    t = f * (e2x * qx + e2y * qy + e2z * qz)
    hit = aok & (u >= 0.0) & (vv >= 0.0) & (u + vv <= 1.0) & (t > 1e-9) & (t <= 1.0)
    hitn = jnp.sum(hit.astype(jnp.int32), axis=1, keepdims=True)
    odd = jnp.bitwise_and(hitn, 1) == 1

    pen = jnp.sum(jnp.where(odd, rdx * rdx + rdy * rdy + rdz * rdz, zero))
    pen_ref[:, :, :] = pen.reshape(1, 1, 1)


@functools.partial(jax.jit, static_argnames=())
def _sr_loss(obj_points, sr_t, face_t):
    g = _N // _TN
    pen_parts, cmap = pl.pallas_call(
        _sr_body,
        grid=(_B, g),
        in_specs=[
            pl.BlockSpec((_TN, 3), lambda b, i: (i, 0)),
            pl.BlockSpec((1, 3, _S), lambda b, i: (b, 0, 0)),
            pl.BlockSpec((1, 9, _F), lambda b, i: (b, 0, 0)),
        ],
        out_specs=[
            pl.BlockSpec((1, 1, 1), lambda b, i: (b * g + i, 0, 0)),
            pl.BlockSpec((_TN, 1), lambda b, i: (i, 0)),
        ],
        out_shape=[
            jax.ShapeDtypeStruct((_B * g, 1, 1), jnp.float32),
            jax.ShapeDtypeStruct((_N, 1), jnp.float32),
        ],
    )(obj_points, sr_t, face_t)
    parts = pen_parts.reshape(_B, g)
    pen = (jnp.sqrt(jnp.sum(parts[0]) + 1e-12)
           + jnp.sqrt(jnp.sum(parts[1]) + 1e-12))
    return pen, cmap.reshape(_N)


def kernel(global_rotation, transl, hand_pose, obj_points, base_verts,
           pose_basis, faces, sample_idx):
    sr_list, face_list = [], []
    for i in range(_B):
        R = _rodrigues(global_rotation[i])
        v = (base_verts + (hand_pose[i] @ pose_basis).reshape(_V, 3)) @ R.T + transl[i]
        fv = v[faces]
        v0 = fv[:, 0]
        e1 = fv[:, 1] - v0
        e2 = fv[:, 2] - v0
        face_list.append(jnp.concatenate([v0, e1, e2], axis=1).T)
        sr_list.append(v[sample_idx].T)
    sr_t = jnp.stack(sr_list)
    face_t = jnp.stack(face_list)
    return _sr_loss(obj_points, sr_t, face_t)


# speedup vs baseline: 2.9701x; 1.0384x over previous
import functools

import jax
import jax.numpy as jnp
from jax.experimental import pallas as pl

_B = 2
_V = 778
_F = 512
_S = 512
_N = 8192
_TN = 512


def _rodrigues(r):
    theta = jnp.linalg.norm(r) + 1e-8
    k = r / theta
    K = jnp.array([[0.0, -k[2], k[1]],
                   [k[2], 0.0, -k[0]],
                   [-k[1], k[0], 0.0]], dtype=r.dtype)
    I = jnp.eye(3, dtype=r.dtype)
    return I + jnp.sin(theta) * K + (1.0 - jnp.cos(theta)) * (K @ K)


def _sr_body(obj_ref, sr_ref, face_ref, pen_ref, cmap_ref):
    ox = obj_ref[:, 0:1]
    oy = obj_ref[:, 1:2]
    oz = obj_ref[:, 2:3]
    srx = sr_ref[0, 0:1, :]
    sry = sr_ref[0, 1:2, :]
    srz = sr_ref[0, 2:3, :]

    dx = ox - srx
    dy = oy - sry
    dz = oz - srz
    d2 = dx * dx + dy * dy + dz * dz
    d2min = jnp.min(d2, axis=1, keepdims=True)
    m = d2 == d2min
    big = jnp.full((), 1e30, jnp.float32)
    zero = jnp.zeros((), jnp.float32)
    tx = jnp.min(jnp.where(m, srx, big), axis=1, keepdims=True)
    ty = jnp.min(jnp.where(m, sry, big), axis=1, keepdims=True)
    tz = jnp.min(jnp.where(m, srz, big), axis=1, keepdims=True)

    cmap_ref[:, :] = 1.0 - 2.0 * (jax.nn.sigmoid(100.0 * d2min) - 0.5)

    rdx = tx - ox
    rdy = ty - oy
    rdz = tz - oz
    v0x = face_ref[0, 0:1, :]
    v0y = face_ref[0, 1:2, :]
    v0z = face_ref[0, 2:3, :]
    e1x = face_ref[0, 3:4, :]
    e1y = face_ref[0, 4:5, :]
    e1z = face_ref[0, 5:6, :]
    e2x = face_ref[0, 6:7, :]
    e2y = face_ref[0, 7:8, :]
    e2z = face_ref[0, 8:9, :]

    hx = rdy * e2z - rdz * e2y
    hy = rdz * e2x - rdx * e2z
    hz = rdx * e2y - rdy * e2x
    a = e1x * hx + e1y * hy + e1z * hz
    aok = jnp.abs(a) > 1e-9
    f = 1.0 / jnp.where(jnp.abs(a) < 1e-9, 1e-9, a)
    sx = ox - v0x
    sy = oy - v0y
    sz = oz - v0z
    u = f * (sx * hx + sy * hy + sz * hz)
    qx = sy * e1z - sz * e1y
    qy = sz * e1x - sx * e1z
    qz = sx * e1y - sy * e1x
    vv = f * (rdx * qx + rdy * qy + rdz * qz)
    t = f * (e2x * qx + e2y * qy + e2z * qz)
    hit = aok & (u >= 0.0) & (vv >= 0.0) & (u + vv <= 1.0) & (t > 1e-9) & (t <= 1.0)
    hitn = jnp.sum(hit.astype(jnp.int32), axis=1, keepdims=True)
    odd = jnp.bitwise_and(hitn, 1) == 1

    pen = jnp.sum(jnp.where(odd, rdx * rdx + rdy * rdy + rdz * rdz, zero))
    pen_ref[:, :, :] = pen.reshape(1, 1, 1)


@functools.partial(jax.jit, static_argnames=())
def _sr_loss(obj_points, sr_t, face_t):
    g = _N // _TN
    pen_parts, cmap = pl.pallas_call(
        _sr_body,
        grid=(_B, g),
        in_specs=[
            pl.BlockSpec((_TN, 3), lambda b, i: (i, 0)),
            pl.BlockSpec((1, 3, _S), lambda b, i: (b, 0, 0)),
            pl.BlockSpec((1, 9, _F), lambda b, i: (b, 0, 0)),
        ],
        out_specs=[
            pl.BlockSpec((1, 1, 1), lambda b, i: (b * g + i, 0, 0)),
            pl.BlockSpec((_TN, 1), lambda b, i: (i, 0)),
        ],
        out_shape=[
            jax.ShapeDtypeStruct((_B * g, 1, 1), jnp.float32),
            jax.ShapeDtypeStruct((_N, 1), jnp.float32),
        ],
    )(obj_points, sr_t, face_t)
    parts = pen_parts.reshape(_B, g)
    pen = (jnp.sqrt(jnp.sum(parts[0]) + 1e-12)
           + jnp.sqrt(jnp.sum(parts[1]) + 1e-12))
    return pen, cmap.reshape(_N)


def kernel(global_rotation, transl, hand_pose, obj_points, base_verts,
           pose_basis, faces, sample_idx):
    sr_list, face_list = [], []
    for i in range(_B):
        R = _rodrigues(global_rotation[i])
        v = (base_verts + (hand_pose[i] @ pose_basis).reshape(_V, 3)) @ R.T + transl[i]
        fv = v[faces]
        v0 = fv[:, 0]
        e1 = fv[:, 1] - v0
        e2 = fv[:, 2] - v0
        face_list.append(jnp.concatenate([v0, e1, e2], axis=1).T)
        sr_list.append(v[sample_idx].T)
    sr_t = jnp.stack(sr_list)
    face_t = jnp.stack(face_list)
    return _sr_loss(obj_points, sr_t, face_t)


# speedup vs baseline: 3.0535x; 1.0281x over previous
import functools

import jax
import jax.numpy as jnp
from jax.experimental import pallas as pl

_B = 2
_V = 778
_F = 512
_S = 512
_N = 8192
_TN = 1024


def _rodrigues(r):
    theta = jnp.linalg.norm(r) + 1e-8
    k = r / theta
    K = jnp.array([[0.0, -k[2], k[1]],
                   [k[2], 0.0, -k[0]],
                   [-k[1], k[0], 0.0]], dtype=r.dtype)
    I = jnp.eye(3, dtype=r.dtype)
    return I + jnp.sin(theta) * K + (1.0 - jnp.cos(theta)) * (K @ K)


def _sr_body(obj_ref, sr_ref, face_ref, pen_ref, cmap_ref):
    ox = obj_ref[:, 0:1]
    oy = obj_ref[:, 1:2]
    oz = obj_ref[:, 2:3]
    srx = sr_ref[0, 0:1, :]
    sry = sr_ref[0, 1:2, :]
    srz = sr_ref[0, 2:3, :]

    dx = ox - srx
    dy = oy - sry
    dz = oz - srz
    d2 = dx * dx + dy * dy + dz * dz
    d2min = jnp.min(d2, axis=1, keepdims=True)
    m = d2 == d2min
    big = jnp.full((), 1e30, jnp.float32)
    zero = jnp.zeros((), jnp.float32)
    tx = jnp.min(jnp.where(m, srx, big), axis=1, keepdims=True)
    ty = jnp.min(jnp.where(m, sry, big), axis=1, keepdims=True)
    tz = jnp.min(jnp.where(m, srz, big), axis=1, keepdims=True)

    cmap_ref[:, :] = 1.0 - 2.0 * (jax.nn.sigmoid(100.0 * d2min) - 0.5)

    rdx = tx - ox
    rdy = ty - oy
    rdz = tz - oz
    v0x = face_ref[0, 0:1, :]
    v0y = face_ref[0, 1:2, :]
    v0z = face_ref[0, 2:3, :]
    e1x = face_ref[0, 3:4, :]
    e1y = face_ref[0, 4:5, :]
    e1z = face_ref[0, 5:6, :]
    e2x = face_ref[0, 6:7, :]
    e2y = face_ref[0, 7:8, :]
    e2z = face_ref[0, 8:9, :]

    hx = rdy * e2z - rdz * e2y
    hy = rdz * e2x - rdx * e2z
    hz = rdx * e2y - rdy * e2x
    a = e1x * hx + e1y * hy + e1z * hz
    aok = jnp.abs(a) > 1e-9
    f = 1.0 / jnp.where(jnp.abs(a) < 1e-9, 1e-9, a)
    sx = ox - v0x
    sy = oy - v0y
    sz = oz - v0z
    u = f * (sx * hx + sy * hy + sz * hz)
    qx = sy * e1z - sz * e1y
    qy = sz * e1x - sx * e1z
    qz = sx * e1y - sy * e1x
    vv = f * (rdx * qx + rdy * qy + rdz * qz)
    t = f * (e2x * qx + e2y * qy + e2z * qz)
    hit = aok & (u >= 0.0) & (vv >= 0.0) & (u + vv <= 1.0) & (t > 1e-9) & (t <= 1.0)
    hitn = jnp.sum(hit.astype(jnp.int32), axis=1, keepdims=True)
    odd = jnp.bitwise_and(hitn, 1) == 1

    pen = jnp.sum(jnp.where(odd, rdx * rdx + rdy * rdy + rdz * rdz, zero))
    pen_ref[:, :, :] = pen.reshape(1, 1, 1)


@functools.partial(jax.jit, static_argnames=())
def _sr_loss(obj_points, sr_t, face_t):
    g = _N // _TN
    pen_parts, cmap = pl.pallas_call(
        _sr_body,
        grid=(_B, g),
        in_specs=[
            pl.BlockSpec((_TN, 3), lambda b, i: (i, 0)),
            pl.BlockSpec((1, 3, _S), lambda b, i: (b, 0, 0)),
            pl.BlockSpec((1, 9, _F), lambda b, i: (b, 0, 0)),
        ],
        out_specs=[
            pl.BlockSpec((1, 1, 1), lambda b, i: (b * g + i, 0, 0)),
            pl.BlockSpec((_TN, 1), lambda b, i: (i, 0)),
        ],
        out_shape=[
            jax.ShapeDtypeStruct((_B * g, 1, 1), jnp.float32),
            jax.ShapeDtypeStruct((_N, 1), jnp.float32),
        ],
    )(obj_points, sr_t, face_t)
    parts = pen_parts.reshape(_B, g)
    pen = (jnp.sqrt(jnp.sum(parts[0]) + 1e-12)
           + jnp.sqrt(jnp.sum(parts[1]) + 1e-12))
    return pen, cmap.reshape(_N)


def kernel(global_rotation, transl, hand_pose, obj_points, base_verts,
           pose_basis, faces, sample_idx):
    sr_list, face_list = [], []
    for i in range(_B):
        R = _rodrigues(global_rotation[i])
        v = (base_verts + (hand_pose[i] @ pose_basis).reshape(_V, 3)) @ R.T + transl[i]
        fv = v[faces]
        v0 = fv[:, 0]
        e1 = fv[:, 1] - v0
        e2 = fv[:, 2] - v0
        face_list.append(jnp.concatenate([v0, e1, e2], axis=1).T)
        sr_list.append(v[sample_idx].T)
    sr_t = jnp.stack(sr_list)
    face_t = jnp.stack(face_list)
    return _sr_loss(obj_points, sr_t, face_t)


# speedup vs baseline: 3.1044x; 1.0167x over previous
import functools

import jax
import jax.numpy as jnp
from jax import lax
from jax.experimental import pallas as pl
from jax.experimental.pallas import tpu as pltpu
from jax.experimental.pallas import tpu_sc as plsc

_B = 2
_V = 778
_F = 512
_S = 512
_N = 8192
_TN = 1024


_NROW = 12


def _sc_gather_body(v_hbm, facest_hbm, sidx_hbm, face_out, sr_out,
                    src_v, src0_v, idx_a, idx_b, val_a, val_b, sem):
    wid = lax.axis_index("s") * 2 + lax.axis_index("c")

    @pl.when(wid < _B * _NROW)
    def _():
        b = wid // _NROW
        r = wid % _NROW
        is_face = r < 9
        vk = r // 3
        c = jnp.where(is_face, r % 3, r - 9)
        k_a = jnp.minimum(vk, 2)
        bbase = b * (_V * 3) + c

        @pl.when(is_face)
        def _():
            pltpu.sync_copy(facest_hbm.at[pl.ds(k_a * _F, _F)], src_v)

        @pl.when(jnp.logical_not(is_face))
        def _():
            pltpu.sync_copy(sidx_hbm, src_v)

        @pl.when(is_face & (vk > 0))
        def _():
            pltpu.sync_copy(facest_hbm.at[pl.ds(0, _F)], src0_v)

        for j in range(4):
            for i in range(8):
                sl = pl.ds(j * 128 + i * 16, 16)
                idx_a[j, pl.ds(i * 16, 16)] = src_v[sl] * 3 + bbase
                idx_b[j, pl.ds(i * 16, 16)] = src0_v[sl] * 3 + bbase

        for j in range(4):
            pltpu.async_copy(v_hbm.at[idx_a.at[j]],
                             val_a.at[pl.ds(j * 128, 128)], sem).wait()

        @pl.when(is_face & (vk > 0))
        def _():
            for j in range(4):
                pltpu.async_copy(v_hbm.at[idx_b.at[j]],
                                 val_b.at[pl.ds(j * 128, 128)], sem).wait()
            for i in range(32):
                sl = pl.ds(i * 16, 16)
                val_a[sl] = val_a[sl] - val_b[sl]

        @pl.when(is_face)
        def _():
            pltpu.sync_copy(val_a, face_out.at[pl.ds((b * 9 + r) * _F, _F)])

        @pl.when(jnp.logical_not(is_face))
        def _():
            pltpu.sync_copy(val_a, sr_out.at[pl.ds((b * 3 + (r - 9)) * _S, _S)])


def _sc_gather(v_flat, faces_flat, sample_idx):
    mesh = plsc.VectorSubcoreMesh(core_axis_name="c", subcore_axis_name="s")
    k = functools.partial(
        pl.kernel, mesh=mesh,
        out_type=[jax.ShapeDtypeStruct((_B * 9 * _F,), jnp.float32),
                  jax.ShapeDtypeStruct((_B * 3 * _S,), jnp.float32)],
        scratch_types=[
            pltpu.VMEM((_F,), jnp.int32),
            pltpu.VMEM((_F,), jnp.int32),
            pltpu.VMEM((4, 128), jnp.int32),
            pltpu.VMEM((4, 128), jnp.int32),
            pltpu.VMEM((_F,), jnp.float32),
            pltpu.VMEM((_F,), jnp.float32),
            pltpu.SemaphoreType.DMA,
        ],
    )(_sc_gather_body)
    face_flat, sr_flat = k(v_flat, faces_flat, sample_idx)
    return face_flat.reshape(_B, 9, _F), sr_flat.reshape(_B, 3, _S)


def _rodrigues(r):
    theta = jnp.linalg.norm(r) + 1e-8
    k = r / theta
    K = jnp.array([[0.0, -k[2], k[1]],
                   [k[2], 0.0, -k[0]],
                   [-k[1], k[0], 0.0]], dtype=r.dtype)
    I = jnp.eye(3, dtype=r.dtype)
    return I + jnp.sin(theta) * K + (1.0 - jnp.cos(theta)) * (K @ K)


def _sr_body(obj_ref, sr_ref, face_ref, pen_ref, cmap_ref):
    ox = obj_ref[:, 0:1]
    oy = obj_ref[:, 1:2]
    oz = obj_ref[:, 2:3]
    srx = sr_ref[0, 0:1, :]
    sry = sr_ref[0, 1:2, :]
    srz = sr_ref[0, 2:3, :]

    dx = ox - srx
    dy = oy - sry
    dz = oz - srz
    d2 = dx * dx + dy * dy + dz * dz
    d2min = jnp.min(d2, axis=1, keepdims=True)
    m = d2 == d2min
    big = jnp.full((), 1e30, jnp.float32)
    zero = jnp.zeros((), jnp.float32)
    tx = jnp.min(jnp.where(m, srx, big), axis=1, keepdims=True)
    ty = jnp.min(jnp.where(m, sry, big), axis=1, keepdims=True)
    tz = jnp.min(jnp.where(m, srz, big), axis=1, keepdims=True)

    cmap_ref[:, :] = 1.0 - 2.0 * (jax.nn.sigmoid(100.0 * d2min) - 0.5)

    rdx = tx - ox
    rdy = ty - oy
    rdz = tz - oz
    v0x = face_ref[0, 0:1, :]
    v0y = face_ref[0, 1:2, :]
    v0z = face_ref[0, 2:3, :]
    e1x = face_ref[0, 3:4, :]
    e1y = face_ref[0, 4:5, :]
    e1z = face_ref[0, 5:6, :]
    e2x = face_ref[0, 6:7, :]
    e2y = face_ref[0, 7:8, :]
    e2z = face_ref[0, 8:9, :]

    hx = rdy * e2z - rdz * e2y
    hy = rdz * e2x - rdx * e2z
    hz = rdx * e2y - rdy * e2x
    a = e1x * hx + e1y * hy + e1z * hz
    aok = jnp.abs(a) > 1e-9
    f = 1.0 / jnp.where(jnp.abs(a) < 1e-9, 1e-9, a)
    sx = ox - v0x
    sy = oy - v0y
    sz = oz - v0z
    u = f * (sx * hx + sy * hy + sz * hz)
    qx = sy * e1z - sz * e1y
    qy = sz * e1x - sx * e1z
    qz = sx * e1y - sy * e1x
    vv = f * (rdx * qx + rdy * qy + rdz * qz)
    t = f * (e2x * qx + e2y * qy + e2z * qz)
    hit = aok & (u >= 0.0) & (vv >= 0.0) & (u + vv <= 1.0) & (t > 1e-9) & (t <= 1.0)
    hitn = jnp.sum(hit.astype(jnp.int32), axis=1, keepdims=True)
    odd = jnp.bitwise_and(hitn, 1) == 1

    pen = jnp.sum(jnp.where(odd, rdx * rdx + rdy * rdy + rdz * rdz, zero))
    pen_ref[:, :, :] = pen.reshape(1, 1, 1)


@functools.partial(jax.jit, static_argnames=())
def _sr_loss(obj_points, sr_t, face_t):
    g = _N // _TN
    pen_parts, cmap = pl.pallas_call(
        _sr_body,
        grid=(_B, g),
        in_specs=[
            pl.BlockSpec((_TN, 3), lambda b, i: (i, 0)),
            pl.BlockSpec((1, 3, _S), lambda b, i: (b, 0, 0)),
            pl.BlockSpec((1, 9, _F), lambda b, i: (b, 0, 0)),
        ],
        out_specs=[
            pl.BlockSpec((1, 1, 1), lambda b, i: (b * g + i, 0, 0)),
            pl.BlockSpec((_TN, 1), lambda b, i: (i, 0)),
        ],
        out_shape=[
            jax.ShapeDtypeStruct((_B * g, 1, 1), jnp.float32),
            jax.ShapeDtypeStruct((_N, 1), jnp.float32),
        ],
    )(obj_points, sr_t, face_t)
    parts = pen_parts.reshape(_B, g)
    pen = (jnp.sqrt(jnp.sum(parts[0]) + 1e-12)
           + jnp.sqrt(jnp.sum(parts[1]) + 1e-12))
    return pen, cmap.reshape(_N)


def kernel(global_rotation, transl, hand_pose, obj_points, base_verts,
           pose_basis, faces, sample_idx):
    v_list = []
    for i in range(_B):
        R = _rodrigues(global_rotation[i])
        v = (base_verts + (hand_pose[i] @ pose_basis).reshape(_V, 3)) @ R.T + transl[i]
        v_list.append(v)
    v_flat = jnp.stack(v_list).reshape(_B * _V * 3)
    face_t, sr_t = _sc_gather(v_flat, faces.T.reshape(3 * _F), sample_idx)
    return _sr_loss(obj_points, sr_t, face_t)


# speedup vs baseline: 3.2115x; 1.0345x over previous
import functools

import jax
import jax.numpy as jnp
from jax import lax
from jax.experimental import pallas as pl
from jax.experimental.pallas import tpu as pltpu
from jax.experimental.pallas import tpu_sc as plsc

_B = 2
_V = 778
_F = 512
_S = 512
_N = 8192
_TN = 1024


_NROW = 12


def _sc_gather_body(v_hbm, facest_hbm, sidx_hbm, face_out, sr_out,
                    src_v, src0_v, idx_a, idx_b, val_a, val_b, sem):
    wid = lax.axis_index("s") * 2 + lax.axis_index("c")

    @pl.when(wid < _B * _NROW)
    def _():
        b = wid // _NROW
        r = wid % _NROW
        is_face = r < 9
        vk = r // 3
        c = jnp.where(is_face, r % 3, r - 9)
        k_a = jnp.minimum(vk, 2)
        bbase = b * (_V * 3) + c

        @pl.when(is_face)
        def _():
            pltpu.sync_copy(facest_hbm.at[pl.ds(k_a * _F, _F)], src_v)

        @pl.when(jnp.logical_not(is_face))
        def _():
            pltpu.sync_copy(sidx_hbm, src_v)

        @pl.when(is_face & (vk > 0))
        def _():
            pltpu.sync_copy(facest_hbm.at[pl.ds(0, _F)], src0_v)

        for j in range(4):
            for i in range(8):
                sl = pl.ds(j * 128 + i * 16, 16)
                idx_a[j, pl.ds(i * 16, 16)] = src_v[sl] * 3 + bbase
                idx_b[j, pl.ds(i * 16, 16)] = src0_v[sl] * 3 + bbase

        for j in range(4):
            pltpu.async_copy(v_hbm.at[idx_a.at[j]],
                             val_a.at[pl.ds(j * 128, 128)], sem).wait()

        @pl.when(is_face & (vk > 0))
        def _():
            for j in range(4):
                pltpu.async_copy(v_hbm.at[idx_b.at[j]],
                                 val_b.at[pl.ds(j * 128, 128)], sem).wait()
            for i in range(32):
                sl = pl.ds(i * 16, 16)
                val_a[sl] = val_a[sl] - val_b[sl]

        @pl.when(is_face)
        def _():
            pltpu.sync_copy(val_a, face_out.at[pl.ds((b * 9 + r) * _F, _F)])

        @pl.when(jnp.logical_not(is_face))
        def _():
            pltpu.sync_copy(val_a, sr_out.at[pl.ds((b * 3 + (r - 9)) * _S, _S)])


def _sc_gather(v_flat, faces_flat, sample_idx):
    mesh = plsc.VectorSubcoreMesh(core_axis_name="c", subcore_axis_name="s")
    k = functools.partial(
        pl.kernel, mesh=mesh,
        out_type=[jax.ShapeDtypeStruct((_B * 9 * _F,), jnp.float32),
                  jax.ShapeDtypeStruct((_B * 3 * _S,), jnp.float32)],
        scratch_types=[
            pltpu.VMEM((_F,), jnp.int32),
            pltpu.VMEM((_F,), jnp.int32),
            pltpu.VMEM((4, 128), jnp.int32),
            pltpu.VMEM((4, 128), jnp.int32),
            pltpu.VMEM((_F,), jnp.float32),
            pltpu.VMEM((_F,), jnp.float32),
            pltpu.SemaphoreType.DMA,
        ],
    )(_sc_gather_body)
    face_flat, sr_flat = k(v_flat, faces_flat, sample_idx)
    return face_flat.reshape(_B, 9, _F), sr_flat.reshape(_B, 3, _S)


def _rodrigues(r):
    theta = jnp.linalg.norm(r) + 1e-8
    k = r / theta
    K = jnp.array([[0.0, -k[2], k[1]],
                   [k[2], 0.0, -k[0]],
                   [-k[1], k[0], 0.0]], dtype=r.dtype)
    I = jnp.eye(3, dtype=r.dtype)
    return I + jnp.sin(theta) * K + (1.0 - jnp.cos(theta)) * (K @ K)


def _sr_body(obj_ref, sr_ref, face_ref, pen_ref, cmap_ref, acc_ref):
    b = pl.program_id(0)
    i = pl.program_id(1)
    g = pl.num_programs(1)
    ox = obj_ref[:, 0:1]
    oy = obj_ref[:, 1:2]
    oz = obj_ref[:, 2:3]
    srx = sr_ref[0, 0:1, :]
    sry = sr_ref[0, 1:2, :]
    srz = sr_ref[0, 2:3, :]

    dx = ox - srx
    dy = oy - sry
    dz = oz - srz
    d2 = dx * dx + dy * dy + dz * dz
    d2min = jnp.min(d2, axis=1, keepdims=True)
    m = d2 == d2min
    big = jnp.full((), 1e30, jnp.float32)
    zero = jnp.zeros((), jnp.float32)
    tx = jnp.min(jnp.where(m, srx, big), axis=1, keepdims=True)
    ty = jnp.min(jnp.where(m, sry, big), axis=1, keepdims=True)
    tz = jnp.min(jnp.where(m, srz, big), axis=1, keepdims=True)

    cmap_ref[:, :] = 1.0 - 2.0 * (jax.nn.sigmoid(100.0 * d2min) - 0.5)

    rdx = tx - ox
    rdy = ty - oy
    rdz = tz - oz
    v0x = face_ref[0, 0:1, :]
    v0y = face_ref[0, 1:2, :]
    v0z = face_ref[0, 2:3, :]
    e1x = face_ref[0, 3:4, :]
    e1y = face_ref[0, 4:5, :]
    e1z = face_ref[0, 5:6, :]
    e2x = face_ref[0, 6:7, :]
    e2y = face_ref[0, 7:8, :]
    e2z = face_ref[0, 8:9, :]

    hx = rdy * e2z - rdz * e2y
    hy = rdz * e2x - rdx * e2z
    hz = rdx * e2y - rdy * e2x
    a = e1x * hx + e1y * hy + e1z * hz
    aok = jnp.abs(a) > 1e-9
    f = 1.0 / jnp.where(jnp.abs(a) < 1e-9, 1e-9, a)
    sx = ox - v0x
    sy = oy - v0y
    sz = oz - v0z
    u = f * (sx * hx + sy * hy + sz * hz)
    qx = sy * e1z - sz * e1y
    qy = sz * e1x - sx * e1z
    qz = sx * e1y - sy * e1x
    vv = f * (rdx * qx + rdy * qy + rdz * qz)
    t = f * (e2x * qx + e2y * qy + e2z * qz)
    hit = aok & (u >= 0.0) & (vv >= 0.0) & (u + vv <= 1.0) & (t > 1e-9) & (t <= 1.0)
    hitn = jnp.sum(hit.astype(jnp.int32), axis=1, keepdims=True)
    odd = jnp.bitwise_and(hitn, 1) == 1

    pen = jnp.sum(jnp.where(odd, rdx * rdx + rdy * rdy + rdz * rdz, zero))

    @pl.when(jnp.logical_and(b == 0, i == 0))
    def _():
        pen_ref[:, :] = jnp.zeros((1, 1), jnp.float32)

    @pl.when(i == 0)
    def _():
        acc_ref[:, :] = jnp.zeros((1, 1), jnp.float32)

    acc_ref[:, :] += pen.reshape(1, 1)

    @pl.when(i == g - 1)
    def _():
        pen_ref[:, :] += jnp.sqrt(acc_ref[:, :] + 1e-12)


@functools.partial(jax.jit, static_argnames=())
def _sr_loss(obj_points, sr_t, face_t):
    g = _N // _TN
    pen_out, cmap = pl.pallas_call(
        _sr_body,
        grid=(_B, g),
        in_specs=[
            pl.BlockSpec((_TN, 3), lambda b, i: (i, 0)),
            pl.BlockSpec((1, 3, _S), lambda b, i: (b, 0, 0)),
            pl.BlockSpec((1, 9, _F), lambda b, i: (b, 0, 0)),
        ],
        out_specs=[
            pl.BlockSpec((1, 1), lambda b, i: (0, 0)),
            pl.BlockSpec((_TN, 1), lambda b, i: (i, 0)),
        ],
        out_shape=[
            jax.ShapeDtypeStruct((1, 1), jnp.float32),
            jax.ShapeDtypeStruct((_N, 1), jnp.float32),
        ],
        scratch_shapes=[pltpu.VMEM((1, 1), jnp.float32)],
    )(obj_points, sr_t, face_t)
    return pen_out.reshape(()), cmap.reshape(_N)


def kernel(global_rotation, transl, hand_pose, obj_points, base_verts,
           pose_basis, faces, sample_idx):
    v_list = []
    for i in range(_B):
        R = _rodrigues(global_rotation[i])
        v = (base_verts + (hand_pose[i] @ pose_basis).reshape(_V, 3)) @ R.T + transl[i]
        v_list.append(v)
    v_flat = jnp.stack(v_list).reshape(_B * _V * 3)
    face_t, sr_t = _sc_gather(v_flat, faces.T.reshape(3 * _F), sample_idx)
    return _sr_loss(obj_points, sr_t, face_t)


# speedup vs baseline: 3.3015x; 1.0280x over previous
import functools

import jax
import jax.numpy as jnp
from jax import lax
from jax.experimental import pallas as pl
from jax.experimental.pallas import tpu as pltpu
from jax.experimental.pallas import tpu_sc as plsc

_B = 2
_V = 778
_F = 512
_S = 512
_N = 8192
_TN = 2048


_NROW = 12


def _sc_gather_body(v_hbm, facest_hbm, sidx_hbm, face_out, sr_out,
                    src_v, src0_v, idx_a, idx_b, val_a, val_b, sem):
    wid = lax.axis_index("s") * 2 + lax.axis_index("c")

    @pl.when(wid < _B * _NROW)
    def _():
        b = wid // _NROW
        r = wid % _NROW
        is_face = r < 9
        vk = r // 3
        c = jnp.where(is_face, r % 3, r - 9)
        k_a = jnp.minimum(vk, 2)
        bbase = b * (_V * 3) + c

        @pl.when(is_face)
        def _():
            pltpu.sync_copy(facest_hbm.at[pl.ds(k_a * _F, _F)], src_v)

        @pl.when(jnp.logical_not(is_face))
        def _():
            pltpu.sync_copy(sidx_hbm, src_v)

        @pl.when(is_face & (vk > 0))
        def _():
            pltpu.sync_copy(facest_hbm.at[pl.ds(0, _F)], src0_v)

        for j in range(4):
            for i in range(8):
                sl = pl.ds(j * 128 + i * 16, 16)
                idx_a[j, pl.ds(i * 16, 16)] = src_v[sl] * 3 + bbase
                idx_b[j, pl.ds(i * 16, 16)] = src0_v[sl] * 3 + bbase

        for j in range(4):
            pltpu.async_copy(v_hbm.at[idx_a.at[j]],
                             val_a.at[pl.ds(j * 128, 128)], sem).wait()

        @pl.when(is_face & (vk > 0))
        def _():
            for j in range(4):
                pltpu.async_copy(v_hbm.at[idx_b.at[j]],
                                 val_b.at[pl.ds(j * 128, 128)], sem).wait()
            for i in range(32):
                sl = pl.ds(i * 16, 16)
                val_a[sl] = val_a[sl] - val_b[sl]

        @pl.when(is_face)
        def _():
            pltpu.sync_copy(val_a, face_out.at[pl.ds((b * 9 + r) * _F, _F)])

        @pl.when(jnp.logical_not(is_face))
        def _():
            pltpu.sync_copy(val_a, sr_out.at[pl.ds((b * 3 + (r - 9)) * _S, _S)])


def _sc_gather(v_flat, faces_flat, sample_idx):
    mesh = plsc.VectorSubcoreMesh(core_axis_name="c", subcore_axis_name="s")
    k = functools.partial(
        pl.kernel, mesh=mesh,
        out_type=[jax.ShapeDtypeStruct((_B * 9 * _F,), jnp.float32),
                  jax.ShapeDtypeStruct((_B * 3 * _S,), jnp.float32)],
        scratch_types=[
            pltpu.VMEM((_F,), jnp.int32),
            pltpu.VMEM((_F,), jnp.int32),
            pltpu.VMEM((4, 128), jnp.int32),
            pltpu.VMEM((4, 128), jnp.int32),
            pltpu.VMEM((_F,), jnp.float32),
            pltpu.VMEM((_F,), jnp.float32),
            pltpu.SemaphoreType.DMA,
        ],
    )(_sc_gather_body)
    face_flat, sr_flat = k(v_flat, faces_flat, sample_idx)
    return face_flat.reshape(_B, 9, _F), sr_flat.reshape(_B, 3, _S)


def _rodrigues(r):
    theta = jnp.linalg.norm(r) + 1e-8
    k = r / theta
    K = jnp.array([[0.0, -k[2], k[1]],
                   [k[2], 0.0, -k[0]],
                   [-k[1], k[0], 0.0]], dtype=r.dtype)
    I = jnp.eye(3, dtype=r.dtype)
    return I + jnp.sin(theta) * K + (1.0 - jnp.cos(theta)) * (K @ K)


def _sr_body(obj_ref, sr_ref, face_ref, pen_ref, cmap_ref, acc_ref):
    b = pl.program_id(0)
    i = pl.program_id(1)
    g = pl.num_programs(1)
    ox = obj_ref[:, 0:1]
    oy = obj_ref[:, 1:2]
    oz = obj_ref[:, 2:3]
    srx = sr_ref[0, 0:1, :]
    sry = sr_ref[0, 1:2, :]
    srz = sr_ref[0, 2:3, :]

    dx = ox - srx
    dy = oy - sry
    dz = oz - srz
    d2 = dx * dx + dy * dy + dz * dz
    d2min = jnp.min(d2, axis=1, keepdims=True)
    m = d2 == d2min
    big = jnp.full((), 1e30, jnp.float32)
    zero = jnp.zeros((), jnp.float32)
    tx = jnp.min(jnp.where(m, srx, big), axis=1, keepdims=True)
    ty = jnp.min(jnp.where(m, sry, big), axis=1, keepdims=True)
    tz = jnp.min(jnp.where(m, srz, big), axis=1, keepdims=True)

    cmap_ref[:, :] = 1.0 - 2.0 * (jax.nn.sigmoid(100.0 * d2min) - 0.5)

    rdx = tx - ox
    rdy = ty - oy
    rdz = tz - oz
    v0x = face_ref[0, 0:1, :]
    v0y = face_ref[0, 1:2, :]
    v0z = face_ref[0, 2:3, :]
    e1x = face_ref[0, 3:4, :]
    e1y = face_ref[0, 4:5, :]
    e1z = face_ref[0, 5:6, :]
    e2x = face_ref[0, 6:7, :]
    e2y = face_ref[0, 7:8, :]
    e2z = face_ref[0, 8:9, :]

    hx = rdy * e2z - rdz * e2y
    hy = rdz * e2x - rdx * e2z
    hz = rdx * e2y - rdy * e2x
    a = e1x * hx + e1y * hy + e1z * hz
    aok = jnp.abs(a) > 1e-9
    f = 1.0 / jnp.where(jnp.abs(a) < 1e-9, 1e-9, a)
    sx = ox - v0x
    sy = oy - v0y
    sz = oz - v0z
    u = f * (sx * hx + sy * hy + sz * hz)
    qx = sy * e1z - sz * e1y
    qy = sz * e1x - sx * e1z
    qz = sx * e1y - sy * e1x
    vv = f * (rdx * qx + rdy * qy + rdz * qz)
    t = f * (e2x * qx + e2y * qy + e2z * qz)
    hit = aok & (u >= 0.0) & (vv >= 0.0) & (u + vv <= 1.0) & (t > 1e-9) & (t <= 1.0)
    hitn = jnp.sum(hit.astype(jnp.int32), axis=1, keepdims=True)
    odd = jnp.bitwise_and(hitn, 1) == 1

    pen = jnp.sum(jnp.where(odd, rdx * rdx + rdy * rdy + rdz * rdz, zero))

    @pl.when(jnp.logical_and(b == 0, i == 0))
    def _():
        pen_ref[:, :] = jnp.zeros((1, 1), jnp.float32)

    @pl.when(i == 0)
    def _():
        acc_ref[:, :] = jnp.zeros((1, 1), jnp.float32)

    acc_ref[:, :] += pen.reshape(1, 1)

    @pl.when(i == g - 1)
    def _():
        pen_ref[:, :] += jnp.sqrt(acc_ref[:, :] + 1e-12)


@functools.partial(jax.jit, static_argnames=())
def _sr_loss(obj_points, sr_t, face_t):
    g = _N // _TN
    pen_out, cmap = pl.pallas_call(
        _sr_body,
        grid=(_B, g),
        in_specs=[
            pl.BlockSpec((_TN, 3), lambda b, i: (i, 0)),
            pl.BlockSpec((1, 3, _S), lambda b, i: (b, 0, 0)),
            pl.BlockSpec((1, 9, _F), lambda b, i: (b, 0, 0)),
        ],
        out_specs=[
            pl.BlockSpec((1, 1), lambda b, i: (0, 0)),
            pl.BlockSpec((_TN, 1), lambda b, i: (i, 0)),
        ],
        out_shape=[
            jax.ShapeDtypeStruct((1, 1), jnp.float32),
            jax.ShapeDtypeStruct((_N, 1), jnp.float32),
        ],
        scratch_shapes=[pltpu.VMEM((1, 1), jnp.float32)],
    )(obj_points, sr_t, face_t)
    return pen_out.reshape(()), cmap.reshape(_N)


def kernel(global_rotation, transl, hand_pose, obj_points, base_verts,
           pose_basis, faces, sample_idx):
    v_list = []
    for i in range(_B):
        R = _rodrigues(global_rotation[i])
        v = (base_verts + (hand_pose[i] @ pose_basis).reshape(_V, 3)) @ R.T + transl[i]
        v_list.append(v)
    v_flat = jnp.stack(v_list).reshape(_B * _V * 3)
    face_t, sr_t = _sc_gather(v_flat, faces.T.reshape(3 * _F), sample_idx)
    return _sr_loss(obj_points, sr_t, face_t)
